# baseline (speedup 1.0000x reference)
import jax
import jax.numpy as jnp
from jax.experimental import pallas as pl

_SEQ_LENS = (2048, 1792, 1536, 1280, 1024, 896, 768, 512)
_D = 1024
_TILE = 128
_MAX_L = 2048
_N_TILES = tuple(L // _TILE for L in _SEQ_LENS)
_GRID_T = _MAX_L // _TILE


def _body(*refs):
    in_refs = refs[:8]
    out_ref = refs[8]
    i = pl.program_id(0)
    t = pl.program_id(1)
    for k, nk in enumerate(_N_TILES):
        @pl.when(jnp.logical_and(i == k, t < nk))
        def _(k=k):
            out_ref[...] = in_refs[k][...]
        @pl.when(jnp.logical_and(i == k, t >= nk))
        def _():
            out_ref[...] = jnp.zeros(out_ref.shape, out_ref.dtype)


def _in_spec(k):
    nk = _N_TILES[k]

    def index_map(s, t):
        tt = jnp.where(s == k, jnp.minimum(t, nk - 1), 0)
        return (0, tt, 0)

    return pl.BlockSpec((1, _TILE, _D), index_map)


def kernel(seq0, seq1, seq2, seq3, seq4, seq5, seq6, seq7):
    seqs = (seq0, seq1, seq2, seq3, seq4, seq5, seq6, seq7)
    out_shape = jax.ShapeDtypeStruct((8, _MAX_L, _D), seq0.dtype)
    return pl.pallas_call(
        _body,
        grid=(8, _GRID_T),
        in_specs=[_in_spec(k) for k in range(8)],
        out_specs=pl.BlockSpec((1, _TILE, _D), lambda s, t: (s, t, 0)),
        out_shape=out_shape,
    )(*seqs)
